# baseline (speedup 1.0000x reference)
import functools

import jax
import jax.numpy as jnp
from jax import lax
from jax.experimental import pallas as pl
from jax.experimental.pallas import tpu as pltpu
from jax.experimental.pallas import tpu_sc as plsc

N = 10000
E = 320000
D = 128
R = 32
NB = 8

NC = 2
NS = 16
NW = NC * NS
CB = 80
SB = 5
NCHS = 25
EPW = SB * NCHS * CB
EPAD = NW * EPW

RPB = 2


def _wcomb_body(wcomp_ref, basisf_ref, out_ref):
    out_ref[...] = jnp.dot(wcomp_ref[...], basisf_ref[...],
                           preferred_element_type=jnp.float32)


def _wcomb(w_comp, basisf):
    return pl.pallas_call(
        _wcomb_body,
        out_shape=jax.ShapeDtypeStruct((R, D * D), jnp.float32),
    )(w_comp, basisf)


def _hall_body(x_ref, w_ref, loopw_ref, bias_ref, hall_ref, base_ref):
    r = pl.program_id(0)
    x = x_ref[...]
    for k in range(RPB):
        h_bf = jnp.dot(x, w_ref[k],
                       preferred_element_type=jnp.float32).astype(jnp.bfloat16)
        lo = lax.bitcast_convert_type(h_bf[:, :D // 2],
                                      jnp.uint16).astype(jnp.uint32)
        hi = lax.bitcast_convert_type(h_bf[:, D // 2:],
                                      jnp.uint16).astype(jnp.uint32)
        packed = lo | (hi << 16)
        hall_ref[pl.ds(k * N, N), :] = lax.bitcast_convert_type(
            packed, jnp.int32)

    @pl.when(r == 0)
    def _():
        base_ref[...] = jnp.dot(
            x, loopw_ref[...], preferred_element_type=jnp.float32
        ) + bias_ref[...]


def _hall(x, w3, loop_w, bias2d):
    return pl.pallas_call(
        _hall_body,
        grid=(R // RPB,),
        in_specs=[
            pl.BlockSpec((N, D), lambda r: (0, 0)),
            pl.BlockSpec((RPB, D, D), lambda r: (r, 0, 0)),
            pl.BlockSpec((D, D), lambda r: (0, 0)),
            pl.BlockSpec((1, D), lambda r: (0, 0)),
        ],
        out_specs=[
            pl.BlockSpec((RPB * N, D // 2), lambda r: (r, 0)),
            pl.BlockSpec((N, D), lambda r: (0, 0)),
        ],
        out_shape=[
            jax.ShapeDtypeStruct((R * N, D // 2), jnp.int32),
            jax.ShapeDtypeStruct((N, D), jnp.float32),
        ],
    )(x, w3, loop_w, bias2d)


def _combine_body_relu(part_ref, base_ref, out_ref):
    s = part_ref[0] + part_ref[1] + base_ref[...]
    out_ref[...] = jnp.maximum(s, 0.0)


def _combine_body(part_ref, base_ref, out_ref):
    out_ref[...] = part_ref[0] + part_ref[1] + base_ref[...]


def _combine(parts, base, relu):
    return pl.pallas_call(
        _combine_body_relu if relu else _combine_body,
        out_shape=jax.ShapeDtypeStruct((N, D), jnp.float32),
    )(parts, base)


_SPLAT_DNUMS = lax.GatherDimensionNumbers(
    offset_dims=(), collapsed_slice_dims=(0,), start_index_map=(0,))

def _scale_rows(packed_v, frows_v, norm_v, i):
    def _grp(g, _3):
        n16 = norm_v[pl.ds(i * CB + g * 16, 16)]
        for l in range(16):
            e = g * 16 + l
            spl = lax.gather(
                n16, jnp.full((16, 1), l, jnp.int32),
                _SPLAT_DNUMS, slice_sizes=(1,),
                mode=lax.GatherScatterMode.PROMISE_IN_BOUNDS)
            for f in range(D // 32):
                v = packed_v[e, pl.ds(f * 16, 16)]
                a = lax.bitcast_convert_type(v << 16, jnp.float32)
                b = lax.bitcast_convert_type(v & jnp.int32(-65536),
                                             jnp.float32)
                frows_v[e, pl.ds(f * 16, 16)] = a * spl
                frows_v[e, pl.ds(D // 2 + f * 16, 16)] = b * spl
        return 0

    lax.fori_loop(0, CB // 16, _grp, 0)


def _sc_scatter_body(hall_hbm, src_hbm, etype_hbm, dst_hbm, norm_hbm,
                     out_hbm, src_v, etype_v, dst_v, norm_v, idx_v,
                     rows0_v, rows1_v, frows_v, acc_sh, sem0, sem1):
    c = lax.axis_index("c")
    s = lax.axis_index("s")
    wid = s * NC + c
    rows = (rows0_v, rows1_v)
    sems = (sem0, sem1)

    zero16 = jnp.zeros((16,), jnp.float32)

    def _zrow(e, _):
        for f in range(D // 16):
            frows_v[e, pl.ds(f * 16, 16)] = zero16
        return 0

    lax.fori_loop(0, CB, _zrow, 0)

    nchunks = N // CB
    for kk in range(8):
        ch = s + NS * kk

        @pl.when(ch < nchunks)
        def _():
            pltpu.sync_copy(frows_v, acc_sh.at[pl.ds(ch * CB, CB)])

    plsc.subcore_barrier()

    def _sb(sb, _):
        pltpu.sync_copy(src_hbm.at[wid, sb], src_v)
        pltpu.sync_copy(etype_hbm.at[wid, sb], etype_v)
        pltpu.sync_copy(dst_hbm.at[wid, sb], dst_v)
        pltpu.sync_copy(norm_hbm.at[wid, sb], norm_v)

        def _idx(i, _2):
            for j in range(CB // 16):
                sl = pl.ds(j * 16, 16)
                idx_v[i, sl] = etype_v[i, sl] * N + src_v[i, sl]
            return 0

        lax.fori_loop(0, NCHS, _idx, 0)

        pltpu.make_async_copy(hall_hbm.at[idx_v.at[0]], rows0_v,
                              sem0).start()

        def _step(i, b):
            @pl.when(i + 1 < NCHS)
            def _():
                pltpu.make_async_copy(hall_hbm.at[idx_v.at[i + 1]],
                                      rows[1 - b], sems[1 - b]).start()

            pltpu.make_async_copy(hall_hbm.at[idx_v.at[i]],
                                  rows[b], sems[b]).wait()
            _scale_rows(rows[b], frows_v, norm_v, i)
            pltpu.sync_copy(frows_v, acc_sh.at[dst_v.at[i]], add=True)

        def _pair(i2, _2):
            for b in range(2):
                _step(i2 * 2 + b, b)
            return 0

        lax.fori_loop(0, NCHS // 2, _pair, 0)
        if NCHS % 2:
            _step(NCHS - 1, (NCHS - 1) % 2)
        return 0

    lax.fori_loop(0, SB, _sb, 0)

    plsc.subcore_barrier()

    for kk in range(8):
        ch = s + NS * kk

        @pl.when(ch < nchunks)
        def _():
            pltpu.sync_copy(acc_sh.at[pl.ds(ch * CB, CB)],
                            out_hbm.at[c, pl.ds(ch * CB, CB)])


@functools.partial(
    pl.kernel,
    out_type=jax.ShapeDtypeStruct((NC, N, D), jnp.float32),
    mesh=plsc.VectorSubcoreMesh(core_axis_name="c", subcore_axis_name="s"),
    scratch_types=[
        pltpu.VMEM((NCHS, CB), jnp.int32),
        pltpu.VMEM((NCHS, CB), jnp.int32),
        pltpu.VMEM((NCHS, CB), jnp.int32),
        pltpu.VMEM((NCHS * CB,), jnp.float32),
        pltpu.VMEM((NCHS, CB), jnp.int32),
        pltpu.VMEM((CB, D // 2), jnp.int32),
        pltpu.VMEM((CB, D // 2), jnp.int32),
        pltpu.VMEM((CB, D), jnp.float32),
        pltpu.VMEM_SHARED((N, D), jnp.float32),
        pltpu.SemaphoreType.DMA,
        pltpu.SemaphoreType.DMA,
    ],
    compiler_params=pltpu.CompilerParams(use_tc_tiling_on_sc=False),
)
def _sc_scatter(hall_hbm, src_hbm, etype_hbm, dst_hbm, norm_hbm, out_hbm,
                src_v, etype_v, dst_v, norm_v, idx_v, rows0_v, rows1_v,
                frows_v, acc_sh, sem0, sem1):
    _sc_scatter_body(hall_hbm, src_hbm, etype_hbm, dst_hbm, norm_hbm,
                     out_hbm, src_v, etype_v, dst_v, norm_v, idx_v,
                     rows0_v, rows1_v, frows_v, acc_sh, sem0, sem1)


def _layer(x, w_comp, basisf, loop_w, bias, srcm, etypem, dstm, normm, relu):
    wflat = _wcomb(w_comp, basisf)
    w3 = wflat.reshape(R, D, D)
    hall, base = _hall(x, w3, loop_w, bias.reshape(1, D))
    parts = _sc_scatter(hall, srcm, etypem, dstm, normm)
    return _combine(parts, base, relu)


def kernel(emb_weight, edge_index, etype, norm,
           basis1, w_comp1, loop1, bias1,
           basis2, w_comp2, loop2, bias2):
    srcm = edge_index[0].astype(jnp.int32).reshape(NW, SB, NCHS, CB)
    dstm = edge_index[1].astype(jnp.int32).reshape(NW, SB, NCHS, CB)
    etypem = etype.astype(jnp.int32).reshape(NW, SB, NCHS, CB)
    normm = norm.reshape(NW, SB, NCHS * CB)
    b1f = basis1.reshape(NB, D * D)
    b2f = basis2.reshape(NB, D * D)
    h = _layer(emb_weight, w_comp1, b1f, loop1, bias1,
               srcm, etypem, dstm, normm, relu=True)
    return _layer(h, w_comp2, b2f, loop2, bias2,
                  srcm, etypem, dstm, normm, relu=False)


# speedup vs baseline: 2.2322x; 2.2322x over previous
import functools

import jax
import jax.numpy as jnp
from jax import lax
from jax.experimental import pallas as pl
from jax.experimental.pallas import tpu as pltpu
from jax.experimental.pallas import tpu_sc as plsc

N = 10000
E = 320000
D = 128
R = 32
NB = 8

NC = 2
NS = 16
NW = NC * NS
CB = 80
SB = 5
NCHS = 25
EPW = SB * NCHS * CB

RPB = 2


def _hall_body(wcomp_ref, x_ref, basis_ref, loopw_ref, bias_ref,
               hall_ref, base_ref):
    r = pl.program_id(0)
    x = x_ref[...]
    for k in range(RPB):
        w = wcomp_ref[r * RPB + k, 0] * basis_ref[0]
        for b in range(1, NB):
            w = w + wcomp_ref[r * RPB + k, b] * basis_ref[b]
        hall_ref[pl.ds(k * N, N), :] = jnp.dot(
            x, w, preferred_element_type=jnp.float32)

    @pl.when(r == 0)
    def _():
        base_ref[...] = jnp.dot(
            x, loopw_ref[...], preferred_element_type=jnp.float32
        ) + bias_ref[...]


def _hall(x, w_comp, basis, loop_w, bias2d):
    return pl.pallas_call(
        _hall_body,
        grid=(R // RPB,),
        in_specs=[
            pl.BlockSpec(memory_space=pltpu.SMEM),
            pl.BlockSpec((N, D), lambda r: (0, 0)),
            pl.BlockSpec((NB, D, D), lambda r: (0, 0, 0)),
            pl.BlockSpec((D, D), lambda r: (0, 0)),
            pl.BlockSpec((1, D), lambda r: (0, 0)),
        ],
        out_specs=[
            pl.BlockSpec((RPB * N, D), lambda r: (r, 0)),
            pl.BlockSpec((N, D), lambda r: (0, 0)),
        ],
        out_shape=[
            jax.ShapeDtypeStruct((R * N, D), jnp.float32),
            jax.ShapeDtypeStruct((N, D), jnp.float32),
        ],
    )(w_comp, x, basis, loop_w, bias2d)


def _combine_body_relu(part_ref, base_ref, out_ref):
    s = part_ref[0] + part_ref[1] + base_ref[...]
    out_ref[...] = jnp.maximum(s, 0.0)


def _combine_body(part_ref, base_ref, out_ref):
    out_ref[...] = part_ref[0] + part_ref[1] + base_ref[...]


def _combine(parts, base, relu):
    return pl.pallas_call(
        _combine_body_relu if relu else _combine_body,
        out_shape=jax.ShapeDtypeStruct((N, D), jnp.float32),
    )(parts, base)


_SPLAT_DNUMS = lax.GatherDimensionNumbers(
    offset_dims=(), collapsed_slice_dims=(0,), start_index_map=(0,))


def _scale_rows(rows_v, norm_v, i):
    def _grp(g, _3):
        n16 = norm_v[i, pl.ds(g * 16, 16)]
        for l in range(16):
            e = g * 16 + l
            spl = lax.gather(
                n16, jnp.full((16, 1), l, jnp.int32),
                _SPLAT_DNUMS, slice_sizes=(1,),
                mode=lax.GatherScatterMode.PROMISE_IN_BOUNDS)
            for f in range(D // 16):
                sl = pl.ds(f * 16, 16)
                rows_v[e, sl] = rows_v[e, sl] * spl
        return 0

    lax.fori_loop(0, CB // 16, _grp, 0)


def _sc_scatter_body(hall_hbm, src_hbm, etype_hbm, dst_hbm, norm_hbm,
                     out_hbm, src_v, etype_v, dst_v, norm_v, idx_v,
                     rows0_v, rows1_v, acc_sh, sem0, sem1, msem):
    c = lax.axis_index("c")
    s = lax.axis_index("s")
    wid = s * NC + c
    rows = (rows0_v, rows1_v)
    sems = (sem0, sem1)

    def _se_copies(sb):
        return (
            pltpu.make_async_copy(src_hbm.at[wid, sb], src_v, msem),
            pltpu.make_async_copy(etype_hbm.at[wid, sb], etype_v, msem),
        )

    def _dn_copies(sb):
        return (
            pltpu.make_async_copy(dst_hbm.at[wid, sb], dst_v, msem),
            pltpu.make_async_copy(norm_hbm.at[wid, sb], norm_v, msem),
        )

    for cp in _se_copies(0) + _dn_copies(0):
        cp.start()

    zero16 = jnp.zeros((16,), jnp.float32)

    def _zrow(e, _):
        for f in range(D // 16):
            rows0_v[e, pl.ds(f * 16, 16)] = zero16
        return 0

    lax.fori_loop(0, CB, _zrow, 0)

    nchunks = N // CB
    for kk in range(8):
        ch = s + NS * kk

        @pl.when(ch < nchunks)
        def _():
            pltpu.sync_copy(rows0_v, acc_sh.at[pl.ds(ch * CB, CB)])

    plsc.subcore_barrier()

    for sb in range(SB):
        if sb > 0:
            for cp in _dn_copies(sb):
                cp.start()
        for cp in _se_copies(sb) + _dn_copies(sb):
            cp.wait()

        def _idx(i, _2):
            for j in range(CB // 16):
                sl = pl.ds(j * 16, 16)
                idx_v[i, sl] = etype_v[i, sl] * N + src_v[i, sl]
            return 0

        lax.fori_loop(0, NCHS, _idx, 0)

        if sb + 1 < SB:
            for cp in _se_copies(sb + 1):
                cp.start()

        pltpu.make_async_copy(hall_hbm.at[idx_v.at[0]], rows0_v,
                              sem0).start()

        def _step(i, b):
            @pl.when(i + 1 < NCHS)
            def _():
                pltpu.make_async_copy(hall_hbm.at[idx_v.at[i + 1]],
                                      rows[1 - b], sems[1 - b]).start()

            pltpu.make_async_copy(hall_hbm.at[idx_v.at[i]],
                                  rows[b], sems[b]).wait()
            _scale_rows(rows[b], norm_v, i)
            pltpu.sync_copy(rows[b], acc_sh.at[dst_v.at[i]], add=True)

        def _pair(i2, _2):
            for b in range(2):
                _step(i2 * 2 + b, b)
            return 0

        lax.fori_loop(0, NCHS // 2, _pair, 0)
        if NCHS % 2:
            _step(NCHS - 1, (NCHS - 1) % 2)

    plsc.subcore_barrier()

    for kk in range(8):
        ch = s + NS * kk

        @pl.when(ch < nchunks)
        def _():
            pltpu.sync_copy(acc_sh.at[pl.ds(ch * CB, CB)],
                            out_hbm.at[c, pl.ds(ch * CB, CB)])


@functools.partial(
    pl.kernel,
    out_type=jax.ShapeDtypeStruct((NC, N, D), jnp.float32),
    mesh=plsc.VectorSubcoreMesh(core_axis_name="c", subcore_axis_name="s"),
    scratch_types=[
        pltpu.VMEM((NCHS, CB), jnp.int32),
        pltpu.VMEM((NCHS, CB), jnp.int32),
        pltpu.VMEM((NCHS, CB), jnp.int32),
        pltpu.VMEM((NCHS, CB), jnp.float32),
        pltpu.VMEM((NCHS, CB), jnp.int32),
        pltpu.VMEM((CB, D), jnp.float32),
        pltpu.VMEM((CB, D), jnp.float32),
        pltpu.VMEM_SHARED((N, D), jnp.float32),
        pltpu.SemaphoreType.DMA,
        pltpu.SemaphoreType.DMA,
        pltpu.SemaphoreType.DMA,
    ],
)
def _sc_scatter(hall_hbm, src_hbm, etype_hbm, dst_hbm, norm_hbm, out_hbm,
                src_v, etype_v, dst_v, norm_v, idx_v, rows0_v, rows1_v,
                acc_sh, sem0, sem1, msem):
    _sc_scatter_body(hall_hbm, src_hbm, etype_hbm, dst_hbm, norm_hbm,
                     out_hbm, src_v, etype_v, dst_v, norm_v, idx_v,
                     rows0_v, rows1_v, acc_sh, sem0, sem1, msem)


def _layer(x, w_comp, basis, loop_w, bias, srcm, etypem, dstm, normm, relu):
    hall, base = _hall(x, w_comp, basis, loop_w, bias.reshape(1, D))
    parts = _sc_scatter(hall, srcm, etypem, dstm, normm)
    return _combine(parts, base, relu)


def kernel(emb_weight, edge_index, etype, norm,
           basis1, w_comp1, loop1, bias1,
           basis2, w_comp2, loop2, bias2):
    srcm = edge_index[0].astype(jnp.int32).reshape(NW, SB, NCHS, CB)
    dstm = edge_index[1].astype(jnp.int32).reshape(NW, SB, NCHS, CB)
    etypem = etype.astype(jnp.int32).reshape(NW, SB, NCHS, CB)
    normm = norm.reshape(NW, SB, NCHS, CB)
    h = _layer(emb_weight, w_comp1, basis1, loop1, bias1,
               srcm, etypem, dstm, normm, relu=True)
    return _layer(h, w_comp2, basis2, loop2, bias2,
                  srcm, etypem, dstm, normm, relu=False)
